# baseline (speedup 1.0000x reference)
import functools

import jax
import jax.numpy as jnp
from jax.experimental import pallas as pl
from jax.experimental.pallas import tpu as pltpu

_CB = 64
_NBL = 128
_KW = 256


def _rb_kernel(*args, H, Bt, NB):
    nv = 4 * NB
    x_ref = args[0]
    w1v = args[1:1 + nv]
    b1_ref = args[1 + nv]
    w2v = args[2 + nv:2 + 2 * nv]
    b2_ref = args[2 + 2 * nv]
    o_ref = args[3 + 2 * nv]
    xA, xB, xC, rA, rB, rC, w1s, w2s = args[4 + 2 * nv:]

    M = Bt * H
    Wl = NB * _NBL

    @pl.when(pl.program_id(0) == 0)
    def _():
        for s in (xA, xB, xC, rA, rB, rC):
            s[...] = jnp.zeros_like(s)
        for ws, wv in ((w1s, w1v), (w2s, w2v)):
            for j in range(NB):
                for r in range(4):
                    rb = 2 * j - 1 + r
                    if 0 <= rb < 2 * NB:
                        ws[j, :, r * _CB:(r + 1) * _CB, :] = (
                            wv[4 * j + r][...].astype(jnp.bfloat16))
                    else:
                        ws[j, :, r * _CB:(r + 1) * _CB, :] = jnp.zeros(
                            (3, _CB, _NBL), jnp.bfloat16)

    for b in range(Bt):
        xv = x_ref[b].astype(jnp.bfloat16)
        lo = b * H
        xB[lo:lo + H, _CB:_CB + Wl] = xv
        xA[lo + 1:lo + H, _CB:_CB + Wl] = xv[0:H - 1]
        xC[lo:lo + H - 1, _CB:_CB + Wl] = xv[1:H]

    for j in range(NB):
        cw = j * _NBL
        a = jnp.dot(xA[:, cw:cw + _KW], w1s[j, 0],
                    preferred_element_type=jnp.float32)
        a += jnp.dot(xB[:, cw:cw + _KW], w1s[j, 1],
                     preferred_element_type=jnp.float32)
        a += jnp.dot(xC[:, cw:cw + _KW], w1s[j, 2],
                     preferred_element_type=jnp.float32)
        r = jnp.maximum(a + b1_ref[:, cw:cw + _NBL], 0.0).astype(jnp.bfloat16)
        r3 = r.reshape(Bt, H, _NBL)
        zrow = jnp.zeros((Bt, 1, _NBL), jnp.bfloat16)
        co = _CB + cw
        rB[:, co:co + _NBL] = r
        rA[:, co:co + _NBL] = jnp.concatenate(
            [zrow, r3[:, 0:H - 1]], axis=1).reshape(M, _NBL)
        rC[:, co:co + _NBL] = jnp.concatenate(
            [r3[:, 1:H], zrow], axis=1).reshape(M, _NBL)

    for j in range(NB):
        cw = j * _NBL
        a = jnp.dot(rA[:, cw:cw + _KW], w2s[j, 0],
                    preferred_element_type=jnp.float32)
        a += jnp.dot(rB[:, cw:cw + _KW], w2s[j, 1],
                     preferred_element_type=jnp.float32)
        a += jnp.dot(rC[:, cw:cw + _KW], w2s[j, 2],
                     preferred_element_type=jnp.float32)
        a3 = (a + b2_ref[:, cw:cw + _NBL]).reshape(Bt, H, _NBL)
        o_ref[:, :, cw:cw + _NBL] = a3


@functools.partial(jax.jit, static_argnames=("bt",))
def _resblock(x_lane, w1_stack, b1_lane, w2_stack, b2_lane, *, bt):
    B, H, Wl = x_lane.shape
    NB = Wl // _NBL
    M = bt * H

    def view_spec(j, r):
        rb = min(max(2 * j - 1 + r, 0), 2 * NB - 1)
        return pl.BlockSpec((3, _CB, _NBL), lambda b, rb=rb, j=j: (0, rb, j))

    vspecs = [view_spec(j, r) for j in range(NB) for r in range(4)]
    body = functools.partial(_rb_kernel, H=H, Bt=bt, NB=NB)
    scr = pltpu.VMEM((M, 2 * _CB + Wl), jnp.bfloat16)
    wscr = pltpu.VMEM((NB, 3, _KW, _NBL), jnp.bfloat16)
    return pl.pallas_call(
        body,
        out_shape=jax.ShapeDtypeStruct((B, H, Wl), jnp.float32),
        grid_spec=pltpu.PrefetchScalarGridSpec(
            num_scalar_prefetch=0,
            grid=(B // bt,),
            in_specs=(
                [pl.BlockSpec((bt, H, Wl), lambda b: (b, 0, 0))]
                + vspecs
                + [pl.BlockSpec((1, Wl), lambda b: (0, 0))]
                + vspecs
                + [pl.BlockSpec((1, Wl), lambda b: (0, 0))]
            ),
            out_specs=pl.BlockSpec((bt, H, Wl), lambda b: (b, 0, 0)),
            scratch_shapes=[scr] * 6 + [wscr] * 2,
        ),
        compiler_params=pltpu.CompilerParams(
            dimension_semantics=("arbitrary",)),
    )(x_lane, *([w1_stack] * (4 * NB)), b1_lane,
      *([w2_stack] * (4 * NB)), b2_lane)


def kernel(x_lane, w1_stack, b1_lane, w2_stack, b2_lane):
    return _resblock(x_lane, w1_stack, b1_lane, w2_stack, b2_lane, bt=16)


# speedup vs baseline: 1.0577x; 1.0577x over previous
import functools

import jax
import jax.numpy as jnp
from jax.experimental import pallas as pl
from jax.experimental.pallas import tpu as pltpu

_CB = 64
_NBL = 128
_KW = 256


def _rb_kernel(*args, H, Bt, NB):
    nv = 4 * NB
    x_ref = args[0]
    w1v = args[1:1 + nv]
    b1_ref = args[1 + nv]
    w2v = args[2 + nv:2 + 2 * nv]
    b2_ref = args[2 + 2 * nv]
    o_ref = args[3 + 2 * nv]
    xA, xB, xC, rA, rB, rC, w1s, w2s = args[4 + 2 * nv:]

    M = Bt * H
    Wl = NB * _NBL

    @pl.when(pl.program_id(0) == 0)
    def _():
        for s in (xA, xB, xC, rA, rB, rC):
            s[...] = jnp.zeros_like(s)
        for ws, wv in ((w1s, w1v), (w2s, w2v)):
            for j in range(NB):
                for r in range(4):
                    rb = 2 * j - 1 + r
                    if 0 <= rb < 2 * NB:
                        ws[j, :, r * _CB:(r + 1) * _CB, :] = (
                            wv[4 * j + r][...].astype(jnp.bfloat16))
                    else:
                        ws[j, :, r * _CB:(r + 1) * _CB, :] = jnp.zeros(
                            (3, _CB, _NBL), jnp.bfloat16)

    for b in range(Bt):
        xv = x_ref[b].astype(jnp.bfloat16)
        lo = b * H
        xB[lo:lo + H, _CB:_CB + Wl] = xv
        xA[lo + 1:lo + H, _CB:_CB + Wl] = xv[0:H - 1]
        xC[lo:lo + H - 1, _CB:_CB + Wl] = xv[1:H]

    for j in range(NB):
        cw = j * _NBL
        a = jnp.dot(xA[:, cw:cw + _KW], w1s[j, 0],
                    preferred_element_type=jnp.float32)
        a += jnp.dot(xB[:, cw:cw + _KW], w1s[j, 1],
                     preferred_element_type=jnp.float32)
        a += jnp.dot(xC[:, cw:cw + _KW], w1s[j, 2],
                     preferred_element_type=jnp.float32)
        r = jnp.maximum(a + b1_ref[:, cw:cw + _NBL], 0.0).astype(jnp.bfloat16)
        r3 = r.reshape(Bt, H, _NBL)
        zrow = jnp.zeros((Bt, 1, _NBL), jnp.bfloat16)
        co = _CB + cw
        rB[:, co:co + _NBL] = r
        rA[:, co:co + _NBL] = jnp.concatenate(
            [zrow, r3[:, 0:H - 1]], axis=1).reshape(M, _NBL)
        rC[:, co:co + _NBL] = jnp.concatenate(
            [r3[:, 1:H], zrow], axis=1).reshape(M, _NBL)

    for j in range(NB):
        cw = j * _NBL
        a = jnp.dot(rA[:, cw:cw + _KW], w2s[j, 0],
                    preferred_element_type=jnp.float32)
        a += jnp.dot(rB[:, cw:cw + _KW], w2s[j, 1],
                     preferred_element_type=jnp.float32)
        a += jnp.dot(rC[:, cw:cw + _KW], w2s[j, 2],
                     preferred_element_type=jnp.float32)
        a3 = (a + b2_ref[:, cw:cw + _NBL]).reshape(Bt, H, _NBL)
        o_ref[:, :, cw:cw + _NBL] = a3


@functools.partial(jax.jit, static_argnames=("bt",))
def _resblock(x_lane, w1_stack, b1_lane, w2_stack, b2_lane, *, bt):
    B, H, Wl = x_lane.shape
    NB = Wl // _NBL
    M = bt * H

    def view_spec(j, r):
        rb = min(max(2 * j - 1 + r, 0), 2 * NB - 1)
        return pl.BlockSpec((3, _CB, _NBL), lambda b, rb=rb, j=j: (0, rb, j))

    vspecs = [view_spec(j, r) for j in range(NB) for r in range(4)]
    body = functools.partial(_rb_kernel, H=H, Bt=bt, NB=NB)
    scr = pltpu.VMEM((M, 2 * _CB + Wl), jnp.bfloat16)
    wscr = pltpu.VMEM((NB, 3, _KW, _NBL), jnp.bfloat16)
    return pl.pallas_call(
        body,
        out_shape=jax.ShapeDtypeStruct((B, H, Wl), jnp.float32),
        grid_spec=pltpu.PrefetchScalarGridSpec(
            num_scalar_prefetch=0,
            grid=(B // bt,),
            in_specs=(
                [pl.BlockSpec((bt, H, Wl), lambda b: (b, 0, 0))]
                + vspecs
                + [pl.BlockSpec((1, Wl), lambda b: (0, 0))]
                + vspecs
                + [pl.BlockSpec((1, Wl), lambda b: (0, 0))]
            ),
            out_specs=pl.BlockSpec((bt, H, Wl), lambda b: (b, 0, 0)),
            scratch_shapes=[scr] * 6 + [wscr] * 2,
        ),
        compiler_params=pltpu.CompilerParams(
            dimension_semantics=("arbitrary",)),
    )(x_lane, *([w1_stack] * (4 * NB)), b1_lane,
      *([w2_stack] * (4 * NB)), b2_lane)


def kernel(x_lane, w1_stack, b1_lane, w2_stack, b2_lane):
    return _resblock(x_lane, w1_stack, b1_lane, w2_stack, b2_lane, bt=64)


# speedup vs baseline: 1.0799x; 1.0209x over previous
import functools

import jax
import jax.numpy as jnp
from jax.experimental import pallas as pl
from jax.experimental.pallas import tpu as pltpu

_CB = 64
_NBL = 128
_KW = 256


def _rb_kernel(*args, H, Bt, NB):
    nv = 4 * NB
    x_ref = args[0]
    w1v = args[1:1 + nv]
    b1_ref = args[1 + nv]
    w2v = args[2 + nv:2 + 2 * nv]
    b2_ref = args[2 + 2 * nv]
    o_ref = args[3 + 2 * nv]
    xA, xB, xC, rA, rB, rC, w1s, w2s = args[4 + 2 * nv:]

    M = Bt * H
    Wl = NB * _NBL

    @pl.when(pl.program_id(0) == 0)
    def _():
        for s in (xA, xB, xC, rA, rB, rC):
            s[...] = jnp.zeros_like(s)
        for ws, wv in ((w1s, w1v), (w2s, w2v)):
            for j in range(NB):
                for r in range(4):
                    rb = 2 * j - 1 + r
                    if 0 <= rb < 2 * NB:
                        ws[j, :, r * _CB:(r + 1) * _CB, :] = (
                            wv[4 * j + r][...].astype(jnp.bfloat16))
                    else:
                        ws[j, :, r * _CB:(r + 1) * _CB, :] = jnp.zeros(
                            (3, _CB, _NBL), jnp.bfloat16)

    for b in range(Bt):
        xv = x_ref[b].astype(jnp.bfloat16)
        lo = b * H
        xB[lo:lo + H, _CB:_CB + Wl] = xv
        xA[lo + 1:lo + H, _CB:_CB + Wl] = xv[0:H - 1]
        xC[lo:lo + H - 1, _CB:_CB + Wl] = xv[1:H]

    for j in range(NB):
        cw = j * _NBL
        a = jnp.dot(xA[:, cw:cw + _KW], w1s[j, 0],
                    preferred_element_type=jnp.float32)
        a += jnp.dot(xB[:, cw:cw + _KW], w1s[j, 1],
                     preferred_element_type=jnp.float32)
        a += jnp.dot(xC[:, cw:cw + _KW], w1s[j, 2],
                     preferred_element_type=jnp.float32)
        r = jnp.maximum(a + b1_ref[:, cw:cw + _NBL], 0.0).astype(jnp.bfloat16)
        r3 = r.reshape(Bt, H, _NBL)
        zrow = jnp.zeros((Bt, 1, _NBL), jnp.bfloat16)
        co = _CB + cw
        rB[:, co:co + _NBL] = r
        rA[:, co:co + _NBL] = jnp.concatenate(
            [zrow, r3[:, 0:H - 1]], axis=1).reshape(M, _NBL)
        rC[:, co:co + _NBL] = jnp.concatenate(
            [r3[:, 1:H], zrow], axis=1).reshape(M, _NBL)

    for j in range(NB):
        cw = j * _NBL
        a = jnp.dot(rA[:, cw:cw + _KW], w2s[j, 0],
                    preferred_element_type=jnp.float32)
        a += jnp.dot(rB[:, cw:cw + _KW], w2s[j, 1],
                     preferred_element_type=jnp.float32)
        a += jnp.dot(rC[:, cw:cw + _KW], w2s[j, 2],
                     preferred_element_type=jnp.float32)
        a3 = (a + b2_ref[:, cw:cw + _NBL]).reshape(Bt, H, _NBL)
        o_ref[:, :, cw:cw + _NBL] = a3


@functools.partial(jax.jit, static_argnames=("bt",))
def _resblock(x_lane, w1_stack, b1_lane, w2_stack, b2_lane, *, bt):
    B, H, Wl = x_lane.shape
    NB = Wl // _NBL
    M = bt * H

    def view_spec(j, r):
        rb = min(max(2 * j - 1 + r, 0), 2 * NB - 1)
        return pl.BlockSpec((3, _CB, _NBL), lambda b, rb=rb, j=j: (0, rb, j))

    vspecs = [view_spec(j, r) for j in range(NB) for r in range(4)]
    body = functools.partial(_rb_kernel, H=H, Bt=bt, NB=NB)
    scr = pltpu.VMEM((M, 2 * _CB + Wl), jnp.bfloat16)
    wscr = pltpu.VMEM((NB, 3, _KW, _NBL), jnp.bfloat16)
    return pl.pallas_call(
        body,
        out_shape=jax.ShapeDtypeStruct((B, H, Wl), jnp.float32),
        grid_spec=pltpu.PrefetchScalarGridSpec(
            num_scalar_prefetch=0,
            grid=(B // bt,),
            in_specs=(
                [pl.BlockSpec((bt, H, Wl), lambda b: (b, 0, 0))]
                + vspecs
                + [pl.BlockSpec((1, Wl), lambda b: (0, 0))]
                + vspecs
                + [pl.BlockSpec((1, Wl), lambda b: (0, 0))]
            ),
            out_specs=pl.BlockSpec((bt, H, Wl), lambda b: (b, 0, 0)),
            scratch_shapes=[scr] * 6 + [wscr] * 2,
        ),
        compiler_params=pltpu.CompilerParams(
            dimension_semantics=("arbitrary",)),
    )(x_lane, *([w1_stack] * (4 * NB)), b1_lane,
      *([w2_stack] * (4 * NB)), b2_lane)


def kernel(x_lane, w1_stack, b1_lane, w2_stack, b2_lane):
    return _resblock(x_lane, w1_stack, b1_lane, w2_stack, b2_lane, bt=32)
